# baseline (speedup 1.0000x reference)
import functools

import jax
import jax.numpy as jnp
from jax import lax
from jax.experimental import pallas as pl
from jax.experimental.pallas import tpu as pltpu
from jax.experimental.pallas import tpu_sc as plsc

NUM_EMBEDDINGS = 8192
D = 768
XROWS = 4096
XCOLS = 50
B = XROWS * XCOLS

NC = 2
NS = 16
NW = NC * NS
R_PER_W = XROWS // NW
B_PER_W = B // NW
CHUNK = 32
NBUF = 4
CPB = R_PER_W // CHUNK
N_CHUNKS = B_PER_W // CHUNK


def _gather_sc(idx_t_flat, table):
    mesh = plsc.VectorSubcoreMesh(core_axis_name="c", subcore_axis_name="s")

    scratch = [pltpu.VMEM((B_PER_W,), jnp.int32)]
    scratch += [pltpu.VMEM((CHUNK, D), jnp.float32) for _ in range(NBUF)]
    scratch += [pltpu.SemaphoreType.DMA]
    scratch += [pltpu.SemaphoreType.DMA for _ in range(2 * NBUF)]

    @functools.partial(
        pl.kernel,
        mesh=mesh,
        out_type=jax.ShapeDtypeStruct((XCOLS, XROWS, D), jnp.float32),
        scratch_types=scratch,
    )
    def k(idx_hbm, table_hbm, out_hbm, idx_v, *bufs_and_sems):
        rows = bufs_and_sems[:NBUF]
        isem = bufs_and_sems[NBUF]
        gsem = bufs_and_sems[NBUF + 1:NBUF + 1 + NBUF]
        ssem = bufs_and_sems[NBUF + 1 + NBUF:]
        wid = lax.axis_index("s") * NC + lax.axis_index("c")
        i0 = wid * R_PER_W

        for j in range(XCOLS):
            pltpu.async_copy(
                idx_hbm.at[pl.ds(j * XROWS + i0, R_PER_W)],
                idx_v.at[pl.ds(j * R_PER_W, R_PER_W)], isem)

        def wait_stage():
            pltpu.make_async_copy(
                idx_hbm.at[pl.ds(0, R_PER_W)],
                idx_v.at[pl.ds(0, R_PER_W)], isem).wait()

        wait_stage()

        def start_gather(b, t):
            pltpu.async_copy(
                table_hbm.at[idx_v.at[pl.ds(t * CHUNK, CHUNK)]], rows[b],
                gsem[b])

        def wait_gather(b):
            pltpu.make_async_copy(
                table_hbm.at[pl.ds(0, CHUNK)], rows[b], gsem[b]).wait()

        def start_store(b, t):
            j = t // CPB
            h = t % CPB
            pltpu.async_copy(
                rows[b], out_hbm.at[j].at[pl.ds(i0 + h * CHUNK, CHUNK)],
                ssem[b])

        def wait_store(b):
            pltpu.make_async_copy(
                rows[b], out_hbm.at[0].at[pl.ds(0, CHUNK)], ssem[b]).wait()

        for b in range(NBUF):
            start_gather(b, b)

        for j in range(1, XCOLS):
            wait_stage()

        @pl.loop(0, N_CHUNKS - NBUF, step=NBUF)
        def _(t):
            for b in range(NBUF):
                wait_gather(b)
                start_store(b, t + b)
            for b in range(NBUF):
                wait_store(b)
                start_gather(b, t + NBUF + b)

        for b in range(NBUF):
            wait_gather(b)
            start_store(b, N_CHUNKS - NBUF + b)
        for b in range(NBUF):
            wait_store(b)

    return k(idx_t_flat, table)


def kernel(x, table):
    idx_t_flat = x.T.reshape(-1)
    out_t = _gather_sc(idx_t_flat, table)
    return jnp.transpose(out_t, (1, 0, 2))
